# baseline (speedup 1.0000x reference)
import jax
import jax.numpy as jnp
from jax.experimental import pallas as pl
from jax.experimental.pallas import tpu as pltpu


def kernel(x, language_id, language_embeddings):
    batch, seq, d = x.shape
    tile = 1024
    tab3 = language_embeddings[:, None, :]
    lid = language_id.astype(jnp.int32)

    def body(lid_ref, x_ref, e_ref, o_ref):
        o_ref[...] = x_ref[...] + e_ref[...]

    grid_spec = pltpu.PrefetchScalarGridSpec(
        num_scalar_prefetch=1,
        grid=(batch, seq // tile),
        in_specs=[
            pl.BlockSpec((1, tile, d), lambda i, j, lid_ref: (i, j, 0)),
            pl.BlockSpec((1, 1, d), lambda i, j, lid_ref: (lid_ref[i], 0, 0)),
        ],
        out_specs=pl.BlockSpec((1, tile, d), lambda i, j, lid_ref: (i, j, 0)),
    )
    return pl.pallas_call(
        body,
        grid_spec=grid_spec,
        out_shape=jax.ShapeDtypeStruct(x.shape, x.dtype),
        compiler_params=pltpu.CompilerParams(
            dimension_semantics=("parallel", "parallel"),
        ),
    )(lid, x, tab3)
